# baseline (speedup 1.0000x reference)
import functools

import jax
import jax.numpy as jnp
from jax import lax
from jax.experimental import pallas as pl
from jax.experimental.pallas import tpu as pltpu
from jax.experimental.pallas import tpu_sc as plsc

VOCAB = 100000
EMBED_DIM = 64
BATCH = 1024


_info = plsc.get_sparse_core_info()
_NC, _NS, _L = _info.num_cores, _info.num_subcores, _info.num_lanes
_NW = _NC * _NS
_B_PER_W = BATCH // _NW


def _sc_gather(table, idx):
    mesh = plsc.VectorSubcoreMesh(core_axis_name="c", subcore_axis_name="s")

    @functools.partial(
        pl.kernel,
        mesh=mesh,
        compiler_params=pltpu.CompilerParams(use_tc_tiling_on_sc=False),
        out_type=jax.ShapeDtypeStruct((BATCH, EMBED_DIM), jnp.float32),
        scratch_types=[
            pltpu.VMEM((_B_PER_W,), jnp.int32),
            pltpu.VMEM((_B_PER_W, EMBED_DIM), jnp.float32),
            pltpu.SemaphoreType.DMA,
        ],
    )
    def gather_kernel(table_hbm, idx_hbm, out_hbm, idx_v, rows_v, sem):
        wid = lax.axis_index("s") * _NC + lax.axis_index("c")
        base = wid * _B_PER_W
        pltpu.sync_copy(idx_hbm.at[pl.ds(base, _B_PER_W)], idx_v)
        pltpu.async_copy(table_hbm.at[idx_v], rows_v, sem).wait()
        pltpu.sync_copy(rows_v, out_hbm.at[pl.ds(base, _B_PER_W)])

    return gather_kernel(table, idx)


_V_TILE = 1024
_N_TILES = (VOCAB + _V_TILE - 1) // _V_TILE
_TAIL = VOCAB - (_N_TILES - 1) * _V_TILE
_NSLOT = 4


def _proj_body(e_ref, w_ref, b_ref, o_hbm, acc, acc_tail, sems):
    j = pl.program_id(0)
    slot = lax.rem(j, _NSLOT)

    @pl.when(j >= _NSLOT)
    def _wait_prev():
        pltpu.make_async_copy(
            acc.at[slot],
            o_hbm.at[:, pl.ds((j - _NSLOT) * _V_TILE, _V_TILE)],
            sems.at[slot],
        ).wait()

    res = lax.dot_general(
        e_ref[...].astype(jnp.bfloat16), w_ref[...].astype(jnp.bfloat16),
        (((1,), (1,)), ((), ())),
        preferred_element_type=jnp.float32,
    ) + b_ref[...]

    @pl.when(j < _N_TILES - 1)
    def _copy_full():
        acc[slot] = res
        pltpu.make_async_copy(
            acc.at[slot],
            o_hbm.at[:, pl.ds(j * _V_TILE, _V_TILE)],
            sems.at[slot],
        ).start()

    @pl.when(j == _N_TILES - 1)
    def _copy_tail_and_drain():
        acc_tail[...] = res[:, :_TAIL]
        pltpu.make_async_copy(
            acc_tail,
            o_hbm.at[:, pl.ds((_N_TILES - 1) * _V_TILE, _TAIL)],
            sems.at[(_N_TILES - 1) % _NSLOT],
        ).start()
        for jj in range(_N_TILES - _NSLOT, _N_TILES):
            s = jj % _NSLOT
            if jj < _N_TILES - 1:
                pltpu.make_async_copy(
                    acc.at[s],
                    o_hbm.at[:, pl.ds(jj * _V_TILE, _V_TILE)],
                    sems.at[s],
                ).wait()
            else:
                pltpu.make_async_copy(
                    acc_tail,
                    o_hbm.at[:, pl.ds(jj * _V_TILE, _TAIL)],
                    sems.at[s],
                ).wait()


def _tc_project(embeds, W, b2d):
    return pl.pallas_call(
        _proj_body,
        grid=(_N_TILES,),
        in_specs=[
            pl.BlockSpec((BATCH, EMBED_DIM), lambda j: (0, 0)),
            pl.BlockSpec((_V_TILE, EMBED_DIM), lambda j: (j, 0)),
            pl.BlockSpec((1, _V_TILE), lambda j: (0, j)),
        ],
        out_specs=pl.BlockSpec(memory_space=pl.ANY),
        out_shape=jax.ShapeDtypeStruct((BATCH, VOCAB), jnp.float32),
        scratch_shapes=[
            pltpu.VMEM((_NSLOT, BATCH, _V_TILE), jnp.float32),
            pltpu.VMEM((BATCH, _TAIL), jnp.float32),
            pltpu.SemaphoreType.DMA((_NSLOT,)),
        ],
        compiler_params=pltpu.CompilerParams(
            dimension_semantics=("arbitrary",),
        ),
    )(embeds, W, b2d)


def kernel(inputs, embeddings, W, b):
    embeds = _sc_gather(embeddings, inputs.astype(jnp.int32))
    return _tc_project(embeds, W, b.reshape(1, VOCAB))


# speedup vs baseline: 1.0106x; 1.0106x over previous
import functools

import jax
import jax.numpy as jnp
from jax import lax
from jax.experimental import pallas as pl
from jax.experimental.pallas import tpu as pltpu
from jax.experimental.pallas import tpu_sc as plsc

VOCAB = 100000
EMBED_DIM = 64
BATCH = 1024


_info = plsc.get_sparse_core_info()
_NC, _NS, _L = _info.num_cores, _info.num_subcores, _info.num_lanes
_NW = _NC * _NS
_B_PER_W = BATCH // _NW


def _sc_gather(table, idx):
    mesh = plsc.VectorSubcoreMesh(core_axis_name="c", subcore_axis_name="s")

    @functools.partial(
        pl.kernel,
        mesh=mesh,
        compiler_params=pltpu.CompilerParams(use_tc_tiling_on_sc=False),
        out_type=jax.ShapeDtypeStruct((BATCH, EMBED_DIM), jnp.float32),
        scratch_types=[
            pltpu.VMEM((_B_PER_W,), jnp.int32),
            pltpu.VMEM((_B_PER_W, EMBED_DIM), jnp.float32),
            pltpu.SemaphoreType.DMA,
        ],
    )
    def gather_kernel(table_hbm, idx_hbm, out_hbm, idx_v, rows_v, sem):
        wid = lax.axis_index("s") * _NC + lax.axis_index("c")
        base = wid * _B_PER_W
        pltpu.sync_copy(idx_hbm.at[pl.ds(base, _B_PER_W)], idx_v)
        pltpu.async_copy(table_hbm.at[idx_v], rows_v, sem).wait()
        pltpu.sync_copy(rows_v, out_hbm.at[pl.ds(base, _B_PER_W)])

    return gather_kernel(table, idx)


_V_TILE = 2048
_N_TILES = (VOCAB + _V_TILE - 1) // _V_TILE
_TAIL = VOCAB - (_N_TILES - 1) * _V_TILE
_NSLOT = 3


def _proj_body(e_ref, w_ref, b_ref, o_hbm, acc, acc_tail, sems):
    j = pl.program_id(0)
    slot = lax.rem(j, _NSLOT)

    @pl.when(j >= _NSLOT)
    def _wait_prev():
        pltpu.make_async_copy(
            acc.at[slot],
            o_hbm.at[:, pl.ds((j - _NSLOT) * _V_TILE, _V_TILE)],
            sems.at[slot],
        ).wait()

    res = lax.dot_general(
        e_ref[...].astype(jnp.bfloat16), w_ref[...].astype(jnp.bfloat16),
        (((1,), (1,)), ((), ())),
        preferred_element_type=jnp.float32,
    ) + b_ref[...]

    @pl.when(j < _N_TILES - 1)
    def _copy_full():
        acc[slot] = res
        pltpu.make_async_copy(
            acc.at[slot],
            o_hbm.at[:, pl.ds(j * _V_TILE, _V_TILE)],
            sems.at[slot],
        ).start()

    @pl.when(j == _N_TILES - 1)
    def _copy_tail_and_drain():
        acc_tail[...] = res[:, :_TAIL]
        pltpu.make_async_copy(
            acc_tail,
            o_hbm.at[:, pl.ds((_N_TILES - 1) * _V_TILE, _TAIL)],
            sems.at[(_N_TILES - 1) % _NSLOT],
        ).start()
        for jj in range(_N_TILES - _NSLOT, _N_TILES):
            s = jj % _NSLOT
            if jj < _N_TILES - 1:
                pltpu.make_async_copy(
                    acc.at[s],
                    o_hbm.at[:, pl.ds(jj * _V_TILE, _V_TILE)],
                    sems.at[s],
                ).wait()
            else:
                pltpu.make_async_copy(
                    acc_tail,
                    o_hbm.at[:, pl.ds(jj * _V_TILE, _TAIL)],
                    sems.at[s],
                ).wait()


def _tc_project(embeds, W, b2d):
    return pl.pallas_call(
        _proj_body,
        grid=(_N_TILES,),
        in_specs=[
            pl.BlockSpec((BATCH, EMBED_DIM), lambda j: (0, 0)),
            pl.BlockSpec((_V_TILE, EMBED_DIM), lambda j: (j, 0)),
            pl.BlockSpec((1, _V_TILE), lambda j: (0, j)),
        ],
        out_specs=pl.BlockSpec(memory_space=pl.ANY),
        out_shape=jax.ShapeDtypeStruct((BATCH, VOCAB), jnp.float32),
        scratch_shapes=[
            pltpu.VMEM((_NSLOT, BATCH, _V_TILE), jnp.float32),
            pltpu.VMEM((BATCH, _TAIL), jnp.float32),
            pltpu.SemaphoreType.DMA((_NSLOT,)),
        ],
        compiler_params=pltpu.CompilerParams(
            dimension_semantics=("arbitrary",),
        ),
    )(embeds, W, b2d)


def kernel(inputs, embeddings, W, b):
    embeds = _sc_gather(embeddings, inputs.astype(jnp.int32))
    return _tc_project(embeds, W, b.reshape(1, VOCAB))


# speedup vs baseline: 1.0256x; 1.0149x over previous
import functools

import jax
import jax.numpy as jnp
from jax import lax
from jax.experimental import pallas as pl
from jax.experimental.pallas import tpu as pltpu
from jax.experimental.pallas import tpu_sc as plsc

VOCAB = 100000
EMBED_DIM = 64
BATCH = 1024


_info = plsc.get_sparse_core_info()
_NC, _NS, _L = _info.num_cores, _info.num_subcores, _info.num_lanes
_NW = _NC * _NS
_B_PER_W = BATCH // _NW


_D_PAD = 128


def _sc_gather(table128, idx):
    mesh = plsc.VectorSubcoreMesh(core_axis_name="c", subcore_axis_name="s")

    @functools.partial(
        pl.kernel,
        mesh=mesh,
        out_type=jax.ShapeDtypeStruct((BATCH, _D_PAD), jnp.float32),
        scratch_types=[
            pltpu.VMEM((_B_PER_W,), jnp.int32),
            pltpu.VMEM((_B_PER_W, _D_PAD), jnp.float32),
            pltpu.SemaphoreType.DMA,
        ],
    )
    def gather_kernel(table_hbm, idx_hbm, out_hbm, idx_v, rows_v, sem):
        wid = lax.axis_index("s") * _NC + lax.axis_index("c")
        base = wid * _B_PER_W
        pltpu.sync_copy(idx_hbm.at[pl.ds(base, _B_PER_W)], idx_v)
        pltpu.async_copy(table_hbm.at[idx_v], rows_v, sem).wait()
        pltpu.sync_copy(rows_v, out_hbm.at[pl.ds(base, _B_PER_W)])

    return gather_kernel(table128, idx)


_V_TILE = 2048
_N_TILES = (VOCAB + _V_TILE - 1) // _V_TILE
_TAIL = VOCAB - (_N_TILES - 1) * _V_TILE
_NSLOT = 3


def _proj_body(e_ref, w_ref, b_ref, o_hbm, acc, acc_tail, sems):
    j = pl.program_id(0)
    slot = lax.rem(j, _NSLOT)

    @pl.when(j >= _NSLOT)
    def _wait_prev():
        pltpu.make_async_copy(
            acc.at[slot],
            o_hbm.at[:, pl.ds((j - _NSLOT) * _V_TILE, _V_TILE)],
            sems.at[slot],
        ).wait()

    e64 = e_ref[...][:, :EMBED_DIM]
    res = lax.dot_general(
        e64.astype(jnp.bfloat16), w_ref[...].astype(jnp.bfloat16),
        (((1,), (1,)), ((), ())),
        preferred_element_type=jnp.float32,
    ) + b_ref[...]

    @pl.when(j < _N_TILES - 1)
    def _copy_full():
        acc[slot] = res
        pltpu.make_async_copy(
            acc.at[slot],
            o_hbm.at[:, pl.ds(j * _V_TILE, _V_TILE)],
            sems.at[slot],
        ).start()

    @pl.when(j == _N_TILES - 1)
    def _copy_tail_and_drain():
        acc_tail[...] = res[:, :_TAIL]
        pltpu.make_async_copy(
            acc_tail,
            o_hbm.at[:, pl.ds((_N_TILES - 1) * _V_TILE, _TAIL)],
            sems.at[(_N_TILES - 1) % _NSLOT],
        ).start()
        for jj in range(_N_TILES - _NSLOT, _N_TILES):
            s = jj % _NSLOT
            if jj < _N_TILES - 1:
                pltpu.make_async_copy(
                    acc.at[s],
                    o_hbm.at[:, pl.ds(jj * _V_TILE, _V_TILE)],
                    sems.at[s],
                ).wait()
            else:
                pltpu.make_async_copy(
                    acc_tail,
                    o_hbm.at[:, pl.ds(jj * _V_TILE, _TAIL)],
                    sems.at[s],
                ).wait()


def _tc_project(embeds, W, b2d):
    return pl.pallas_call(
        _proj_body,
        grid=(_N_TILES,),
        in_specs=[
            pl.BlockSpec((BATCH, _D_PAD), lambda j: (0, 0)),
            pl.BlockSpec((_V_TILE, EMBED_DIM), lambda j: (j, 0)),
            pl.BlockSpec((1, _V_TILE), lambda j: (0, j)),
        ],
        out_specs=pl.BlockSpec(memory_space=pl.ANY),
        out_shape=jax.ShapeDtypeStruct((BATCH, VOCAB), jnp.float32),
        scratch_shapes=[
            pltpu.VMEM((_NSLOT, BATCH, _V_TILE), jnp.float32),
            pltpu.VMEM((BATCH, _TAIL), jnp.float32),
            pltpu.SemaphoreType.DMA((_NSLOT,)),
        ],
        compiler_params=pltpu.CompilerParams(
            dimension_semantics=("arbitrary",),
        ),
    )(embeds, W, b2d)


def kernel(inputs, embeddings, W, b):
    table128 = jnp.pad(embeddings, ((0, 0), (0, _D_PAD - EMBED_DIM)))
    embeds128 = _sc_gather(table128, inputs.astype(jnp.int32))
    return _tc_project(embeds128, W, b.reshape(1, VOCAB))


# speedup vs baseline: 1.0275x; 1.0018x over previous
import functools

import jax
import jax.numpy as jnp
from jax import lax
from jax.experimental import pallas as pl
from jax.experimental.pallas import tpu as pltpu
from jax.experimental.pallas import tpu_sc as plsc

VOCAB = 100000
EMBED_DIM = 64
BATCH = 1024


_info = plsc.get_sparse_core_info()
_NC, _NS, _L = _info.num_cores, _info.num_subcores, _info.num_lanes
_NW = _NC * _NS
_B_PER_W = BATCH // _NW


_D_PAD = 128


def _sc_gather(table128, idx):
    mesh = plsc.VectorSubcoreMesh(core_axis_name="c", subcore_axis_name="s")

    @functools.partial(
        pl.kernel,
        mesh=mesh,
        compiler_params=pltpu.CompilerParams(use_tc_tiling_on_sc=True),
        out_type=jax.ShapeDtypeStruct((BATCH, _D_PAD), jnp.float32),
        scratch_types=[
            pltpu.VMEM((_B_PER_W,), jnp.int32),
            pltpu.VMEM((_B_PER_W, _D_PAD), jnp.float32),
            pltpu.SemaphoreType.DMA,
        ],
    )
    def gather_kernel(table_hbm, idx_hbm, out_hbm, idx_v, rows_v, sem):
        wid = lax.axis_index("s") * _NC + lax.axis_index("c")
        base = wid * _B_PER_W
        pltpu.sync_copy(idx_hbm.at[pl.ds(base, _B_PER_W)], idx_v)
        pltpu.async_copy(table_hbm.at[idx_v], rows_v, sem).wait()
        pltpu.sync_copy(rows_v, out_hbm.at[pl.ds(base, _B_PER_W)])

    return gather_kernel(table128, idx)


_V_TILE = 2048
_N_TILES = (VOCAB + _V_TILE - 1) // _V_TILE
_TAIL = VOCAB - (_N_TILES - 1) * _V_TILE
_NSLOT = 3


def _proj_body(e_ref, w_ref, b_ref, o_hbm, acc, acc_tail, sems):
    j = pl.program_id(0)
    slot = lax.rem(j, _NSLOT)

    @pl.when(j >= _NSLOT)
    def _wait_prev():
        pltpu.make_async_copy(
            acc.at[slot],
            o_hbm.at[:, pl.ds((j - _NSLOT) * _V_TILE, _V_TILE)],
            sems.at[slot],
        ).wait()

    e64 = e_ref[...][:, :EMBED_DIM]
    res = lax.dot_general(
        e64.astype(jnp.bfloat16), w_ref[...].astype(jnp.bfloat16),
        (((1,), (1,)), ((), ())),
        preferred_element_type=jnp.float32,
    ) + b_ref[...]

    @pl.when(j < _N_TILES - 1)
    def _copy_full():
        acc[slot] = res
        pltpu.make_async_copy(
            acc.at[slot],
            o_hbm.at[:, pl.ds(j * _V_TILE, _V_TILE)],
            sems.at[slot],
        ).start()

    @pl.when(j == _N_TILES - 1)
    def _copy_tail_and_drain():
        acc_tail[...] = res[:, :_TAIL]
        pltpu.make_async_copy(
            acc_tail,
            o_hbm.at[:, pl.ds((_N_TILES - 1) * _V_TILE, _TAIL)],
            sems.at[(_N_TILES - 1) % _NSLOT],
        ).start()
        for jj in range(_N_TILES - _NSLOT, _N_TILES):
            s = jj % _NSLOT
            if jj < _N_TILES - 1:
                pltpu.make_async_copy(
                    acc.at[s],
                    o_hbm.at[:, pl.ds(jj * _V_TILE, _V_TILE)],
                    sems.at[s],
                ).wait()
            else:
                pltpu.make_async_copy(
                    acc_tail,
                    o_hbm.at[:, pl.ds(jj * _V_TILE, _TAIL)],
                    sems.at[s],
                ).wait()


def _tc_project(embeds, W, b2d):
    return pl.pallas_call(
        _proj_body,
        grid=(_N_TILES,),
        in_specs=[
            pl.BlockSpec((BATCH, _D_PAD), lambda j: (0, 0)),
            pl.BlockSpec((_V_TILE, EMBED_DIM), lambda j: (j, 0)),
            pl.BlockSpec((1, _V_TILE), lambda j: (0, j)),
        ],
        out_specs=pl.BlockSpec(memory_space=pl.ANY),
        out_shape=jax.ShapeDtypeStruct((BATCH, VOCAB), jnp.float32),
        scratch_shapes=[
            pltpu.VMEM((_NSLOT, BATCH, _V_TILE), jnp.float32),
            pltpu.VMEM((BATCH, _TAIL), jnp.float32),
            pltpu.SemaphoreType.DMA((_NSLOT,)),
        ],
        compiler_params=pltpu.CompilerParams(
            dimension_semantics=("arbitrary",),
        ),
    )(embeds, W, b2d)


def kernel(inputs, embeddings, W, b):
    table128 = jnp.pad(embeddings, ((0, 0), (0, _D_PAD - EMBED_DIM)))
    embeds128 = _sc_gather(table128, inputs.astype(jnp.int32))
    return _tc_project(embeds128, W, b.reshape(1, VOCAB))


# speedup vs baseline: 3.0950x; 3.0121x over previous
import functools

import jax
import jax.numpy as jnp
from jax import lax
from jax.experimental import pallas as pl
from jax.experimental.pallas import tpu as pltpu
from jax.experimental.pallas import tpu_sc as plsc

VOCAB = 100000
EMBED_DIM = 64
BATCH = 1024
_D_PAD = 128


_T_TILE = 2048


def _transpose_body(tin_ref, tout_ref):
    t = tin_ref[...]
    tt = jnp.transpose(t, (1, 0))
    tout_ref[...] = jnp.concatenate(
        [tt, jnp.zeros((_T_TILE, _D_PAD - EMBED_DIM), jnp.float32)], axis=1)


def _tc_transpose_pad(tableT):
    n = pl.cdiv(VOCAB, _T_TILE)
    return pl.pallas_call(
        _transpose_body,
        grid=(n,),
        in_specs=[pl.BlockSpec((EMBED_DIM, _T_TILE), lambda j: (0, j))],
        out_specs=pl.BlockSpec((_T_TILE, _D_PAD), lambda j: (j, 0)),
        out_shape=jax.ShapeDtypeStruct((VOCAB, _D_PAD), jnp.float32),
        compiler_params=pltpu.CompilerParams(
            dimension_semantics=("arbitrary",),
        ),
    )(tableT)


_info = plsc.get_sparse_core_info()
_NC, _NS, _L = _info.num_cores, _info.num_subcores, _info.num_lanes
_NW = _NC * _NS
_B_PER_W = BATCH // _NW


def _sc_gather(table128, idx):
    mesh = plsc.VectorSubcoreMesh(core_axis_name="c", subcore_axis_name="s")

    @functools.partial(
        pl.kernel,
        mesh=mesh,
        compiler_params=pltpu.CompilerParams(use_tc_tiling_on_sc=True),
        out_type=jax.ShapeDtypeStruct((BATCH, _D_PAD), jnp.float32),
        scratch_types=[
            pltpu.VMEM((_B_PER_W,), jnp.int32),
            pltpu.VMEM((_B_PER_W, _D_PAD), jnp.float32),
            pltpu.SemaphoreType.DMA,
        ],
    )
    def gather_kernel(table_hbm, idx_hbm, out_hbm, idx_v, rows_v, sem):
        wid = lax.axis_index("s") * _NC + lax.axis_index("c")
        base = wid * _B_PER_W
        pltpu.sync_copy(idx_hbm.at[pl.ds(base, _B_PER_W)], idx_v)
        pltpu.async_copy(table_hbm.at[idx_v], rows_v, sem).wait()
        pltpu.sync_copy(rows_v, out_hbm.at[pl.ds(base, _B_PER_W)])

    return gather_kernel(table128, idx)


_V_TILE = 2048


def _proj_body(wt_ref, e_ref, b_ref, o_ref):
    e64 = e_ref[...][:, :EMBED_DIM].astype(jnp.bfloat16)
    wt = wt_ref[...].astype(jnp.bfloat16)
    res = lax.dot_general(
        wt, e64,
        (((0,), (1,)), ((), ())),
        preferred_element_type=jnp.float32,
    )
    bcol = jnp.transpose(b_ref[...], (1, 0))
    o_ref[...] = res + bcol


def _tc_project_t(WT, e128, b2d):
    n = pl.cdiv(VOCAB, _V_TILE)
    return pl.pallas_call(
        _proj_body,
        grid=(n,),
        in_specs=[
            pl.BlockSpec((EMBED_DIM, _V_TILE), lambda j: (0, j)),
            pl.BlockSpec((BATCH, _D_PAD), lambda j: (0, 0)),
            pl.BlockSpec((1, _V_TILE), lambda j: (0, j)),
        ],
        out_specs=pl.BlockSpec((_V_TILE, BATCH), lambda j: (j, 0)),
        out_shape=jax.ShapeDtypeStruct((VOCAB, BATCH), jnp.float32),
        compiler_params=pltpu.CompilerParams(
            dimension_semantics=("arbitrary",),
        ),
    )(WT, e128, b2d)


def kernel(inputs, embeddings, W, b):
    tableT = embeddings.T
    table128 = _tc_transpose_pad(tableT)
    e128 = _sc_gather(table128, inputs.astype(jnp.int32))
    outT = _tc_project_t(W.T, e128, b.reshape(1, VOCAB))
    return outT.T


# speedup vs baseline: 3.1194x; 1.0079x over previous
import functools

import jax
import jax.numpy as jnp
from jax import lax
from jax.experimental import pallas as pl
from jax.experimental.pallas import tpu as pltpu
from jax.experimental.pallas import tpu_sc as plsc

VOCAB = 100000
EMBED_DIM = 64
BATCH = 1024
_D_PAD = 128


_T_TILE = 2048


def _transpose_body(tin_ref, tout_ref):
    t = tin_ref[...]
    tt = jnp.transpose(t, (1, 0))
    tout_ref[...] = jnp.concatenate(
        [tt, jnp.zeros((_T_TILE, _D_PAD - EMBED_DIM), jnp.float32)], axis=1)


def _tc_transpose_pad(tableT):
    n = pl.cdiv(VOCAB, _T_TILE)
    return pl.pallas_call(
        _transpose_body,
        grid=(n,),
        in_specs=[pl.BlockSpec((EMBED_DIM, _T_TILE), lambda j: (0, j))],
        out_specs=pl.BlockSpec((_T_TILE, _D_PAD), lambda j: (j, 0)),
        out_shape=jax.ShapeDtypeStruct((VOCAB, _D_PAD), jnp.float32),
        compiler_params=pltpu.CompilerParams(
            dimension_semantics=("arbitrary",),
        ),
    )(tableT)


_info = plsc.get_sparse_core_info()
_NC, _NS, _L = _info.num_cores, _info.num_subcores, _info.num_lanes
_NW = _NC * _NS
_B_PER_W = BATCH // _NW


def _sc_gather(table128, idx):
    mesh = plsc.VectorSubcoreMesh(core_axis_name="c", subcore_axis_name="s")

    @functools.partial(
        pl.kernel,
        mesh=mesh,
        compiler_params=pltpu.CompilerParams(use_tc_tiling_on_sc=True),
        out_type=jax.ShapeDtypeStruct((BATCH, _D_PAD), jnp.float32),
        scratch_types=[
            pltpu.VMEM((_B_PER_W,), jnp.int32),
            pltpu.VMEM((_B_PER_W, _D_PAD), jnp.float32),
            pltpu.SemaphoreType.DMA,
        ],
    )
    def gather_kernel(table_hbm, idx_hbm, out_hbm, idx_v, rows_v, sem):
        wid = lax.axis_index("s") * _NC + lax.axis_index("c")
        base = wid * _B_PER_W
        pltpu.sync_copy(idx_hbm.at[pl.ds(base, _B_PER_W)], idx_v)
        pltpu.async_copy(table_hbm.at[idx_v], rows_v, sem).wait()
        pltpu.sync_copy(rows_v, out_hbm.at[pl.ds(base, _B_PER_W)])

    return gather_kernel(table128, idx)


_V_TILE = 4096


def _proj_body(wt_ref, e_ref, b_ref, o_ref):
    e64 = e_ref[...][:, :EMBED_DIM].astype(jnp.bfloat16)
    wt = wt_ref[...].astype(jnp.bfloat16)
    res = lax.dot_general(
        wt, e64,
        (((0,), (1,)), ((), ())),
        preferred_element_type=jnp.float32,
    )
    bcol = jnp.transpose(b_ref[...], (1, 0))
    o_ref[...] = res + bcol


def _tc_project_t(WT, e128, b2d):
    n = pl.cdiv(VOCAB, _V_TILE)
    return pl.pallas_call(
        _proj_body,
        grid=(n,),
        in_specs=[
            pl.BlockSpec((EMBED_DIM, _V_TILE), lambda j: (0, j)),
            pl.BlockSpec((BATCH, _D_PAD), lambda j: (0, 0)),
            pl.BlockSpec((1, _V_TILE), lambda j: (0, j)),
        ],
        out_specs=pl.BlockSpec((_V_TILE, BATCH), lambda j: (j, 0)),
        out_shape=jax.ShapeDtypeStruct((VOCAB, BATCH), jnp.float32),
        compiler_params=pltpu.CompilerParams(
            dimension_semantics=("arbitrary",),
        ),
    )(WT, e128, b2d)


def kernel(inputs, embeddings, W, b):
    tableT = embeddings.T
    table128 = _tc_transpose_pad(tableT)
    e128 = _sc_gather(table128, inputs.astype(jnp.int32))
    outT = _tc_project_t(W.T, e128, b.reshape(1, VOCAB))
    return outT.T


# speedup vs baseline: 3.4605x; 1.1094x over previous
import functools

import jax
import jax.numpy as jnp
from jax import lax
from jax.experimental import pallas as pl
from jax.experimental.pallas import tpu as pltpu
from jax.experimental.pallas import tpu_sc as plsc

VOCAB = 100000
EMBED_DIM = 64
BATCH = 1024
_D_PAD = 128


_T_TILE = 8192


def _transpose_body(tin_ref, tout_ref):
    t = tin_ref[...]
    tt = jnp.transpose(t, (1, 0))
    tout_ref[...] = jnp.concatenate(
        [tt, jnp.zeros((_T_TILE, _D_PAD - EMBED_DIM), jnp.float32)], axis=1)


def _tc_transpose_pad(tableT):
    n = pl.cdiv(VOCAB, _T_TILE)
    return pl.pallas_call(
        _transpose_body,
        grid=(n,),
        in_specs=[pl.BlockSpec((EMBED_DIM, _T_TILE), lambda j: (0, j))],
        out_specs=pl.BlockSpec((_T_TILE, _D_PAD), lambda j: (j, 0)),
        out_shape=jax.ShapeDtypeStruct((VOCAB, _D_PAD), jnp.float32),
        compiler_params=pltpu.CompilerParams(
            dimension_semantics=("arbitrary",),
        ),
    )(tableT)


_info = plsc.get_sparse_core_info()
_NC, _NS, _L = _info.num_cores, _info.num_subcores, _info.num_lanes
_NW = _NC * _NS
_B_PER_W = BATCH // _NW


def _sc_gather(table128, idx):
    mesh = plsc.VectorSubcoreMesh(core_axis_name="c", subcore_axis_name="s")

    @functools.partial(
        pl.kernel,
        mesh=mesh,
        compiler_params=pltpu.CompilerParams(use_tc_tiling_on_sc=True),
        out_type=jax.ShapeDtypeStruct((BATCH, _D_PAD), jnp.float32),
        scratch_types=[
            pltpu.VMEM((_B_PER_W,), jnp.int32),
            pltpu.VMEM((_B_PER_W, _D_PAD), jnp.float32),
            pltpu.SemaphoreType.DMA,
        ],
    )
    def gather_kernel(table_hbm, idx_hbm, out_hbm, idx_v, rows_v, sem):
        wid = lax.axis_index("s") * _NC + lax.axis_index("c")
        base = wid * _B_PER_W
        pltpu.sync_copy(idx_hbm.at[pl.ds(base, _B_PER_W)], idx_v)
        pltpu.async_copy(table_hbm.at[idx_v], rows_v, sem).wait()
        pltpu.sync_copy(rows_v, out_hbm.at[pl.ds(base, _B_PER_W)])

    return gather_kernel(table128, idx)


_V_TILE = 4096


def _proj_body(wt_ref, e_ref, b_ref, o_ref):
    e64 = e_ref[...][:, :EMBED_DIM].astype(jnp.bfloat16)
    wt = wt_ref[...].astype(jnp.bfloat16)
    res = lax.dot_general(
        wt, e64,
        (((0,), (1,)), ((), ())),
        preferred_element_type=jnp.float32,
    )
    bcol = jnp.transpose(b_ref[...], (1, 0))
    o_ref[...] = res + bcol


def _tc_project_t(WT, e128, b2d):
    n = pl.cdiv(VOCAB, _V_TILE)
    return pl.pallas_call(
        _proj_body,
        grid=(n,),
        in_specs=[
            pl.BlockSpec((EMBED_DIM, _V_TILE), lambda j: (0, j)),
            pl.BlockSpec((BATCH, _D_PAD), lambda j: (0, 0)),
            pl.BlockSpec((1, _V_TILE), lambda j: (0, j)),
        ],
        out_specs=pl.BlockSpec((_V_TILE, BATCH), lambda j: (j, 0)),
        out_shape=jax.ShapeDtypeStruct((VOCAB, BATCH), jnp.float32),
        compiler_params=pltpu.CompilerParams(
            dimension_semantics=("arbitrary",),
        ),
    )(WT, e128, b2d)


def kernel(inputs, embeddings, W, b):
    tableT = embeddings.T
    table128 = _tc_transpose_pad(tableT)
    e128 = _sc_gather(table128, inputs.astype(jnp.int32))
    outT = _tc_project_t(W.T, e128, b.reshape(1, VOCAB))
    return outT.T


# speedup vs baseline: 3.4972x; 1.0106x over previous
import functools

import jax
import jax.numpy as jnp
from jax import lax
from jax.experimental import pallas as pl
from jax.experimental.pallas import tpu as pltpu
from jax.experimental.pallas import tpu_sc as plsc

VOCAB = 100000
EMBED_DIM = 64
BATCH = 1024
_D_PAD = 128


_T_TILE = 16384


def _transpose_body(tin_ref, tout_ref):
    t = tin_ref[...]
    tt = jnp.transpose(t, (1, 0))
    tout_ref[...] = jnp.concatenate(
        [tt, jnp.zeros((_T_TILE, _D_PAD - EMBED_DIM), jnp.float32)], axis=1)


def _tc_transpose_pad(tableT):
    n = pl.cdiv(VOCAB, _T_TILE)
    return pl.pallas_call(
        _transpose_body,
        grid=(n,),
        in_specs=[pl.BlockSpec((EMBED_DIM, _T_TILE), lambda j: (0, j))],
        out_specs=pl.BlockSpec((_T_TILE, _D_PAD), lambda j: (j, 0)),
        out_shape=jax.ShapeDtypeStruct((VOCAB, _D_PAD), jnp.float32),
        compiler_params=pltpu.CompilerParams(
            dimension_semantics=("arbitrary",),
        ),
    )(tableT)


_info = plsc.get_sparse_core_info()
_NC, _NS, _L = _info.num_cores, _info.num_subcores, _info.num_lanes
_NW = _NC * _NS
_B_PER_W = BATCH // _NW


def _sc_gather(table128, idx):
    mesh = plsc.VectorSubcoreMesh(core_axis_name="c", subcore_axis_name="s")

    @functools.partial(
        pl.kernel,
        mesh=mesh,
        compiler_params=pltpu.CompilerParams(use_tc_tiling_on_sc=True),
        out_type=jax.ShapeDtypeStruct((BATCH, _D_PAD), jnp.float32),
        scratch_types=[
            pltpu.VMEM((_B_PER_W,), jnp.int32),
            pltpu.VMEM((_B_PER_W, _D_PAD), jnp.float32),
            pltpu.SemaphoreType.DMA,
        ],
    )
    def gather_kernel(table_hbm, idx_hbm, out_hbm, idx_v, rows_v, sem):
        wid = lax.axis_index("s") * _NC + lax.axis_index("c")
        base = wid * _B_PER_W
        pltpu.sync_copy(idx_hbm.at[pl.ds(base, _B_PER_W)], idx_v)
        pltpu.async_copy(table_hbm.at[idx_v], rows_v, sem).wait()
        pltpu.sync_copy(rows_v, out_hbm.at[pl.ds(base, _B_PER_W)])

    return gather_kernel(table128, idx)


_V_TILE = 4096


def _proj_body(wt_ref, e_ref, b_ref, o_ref):
    e64 = e_ref[...][:, :EMBED_DIM].astype(jnp.bfloat16)
    wt = wt_ref[...].astype(jnp.bfloat16)
    res = lax.dot_general(
        wt, e64,
        (((0,), (1,)), ((), ())),
        preferred_element_type=jnp.float32,
    )
    bcol = jnp.transpose(b_ref[...], (1, 0))
    o_ref[...] = res + bcol


def _tc_project_t(WT, e128, b2d):
    n = pl.cdiv(VOCAB, _V_TILE)
    return pl.pallas_call(
        _proj_body,
        grid=(n,),
        in_specs=[
            pl.BlockSpec((EMBED_DIM, _V_TILE), lambda j: (0, j)),
            pl.BlockSpec((BATCH, _D_PAD), lambda j: (0, 0)),
            pl.BlockSpec((1, _V_TILE), lambda j: (0, j)),
        ],
        out_specs=pl.BlockSpec((_V_TILE, BATCH), lambda j: (j, 0)),
        out_shape=jax.ShapeDtypeStruct((VOCAB, BATCH), jnp.float32),
        compiler_params=pltpu.CompilerParams(
            dimension_semantics=("arbitrary",),
        ),
    )(WT, e128, b2d)


def kernel(inputs, embeddings, W, b):
    tableT = embeddings.T
    table128 = _tc_transpose_pad(tableT)
    e128 = _sc_gather(table128, inputs.astype(jnp.int32))
    outT = _tc_project_t(W.T, e128, b.reshape(1, VOCAB))
    return outT.T
